# baseline (speedup 1.0000x reference)
import functools

import jax
import jax.numpy as jnp
from jax import lax
from jax.experimental import pallas as pl
from jax.experimental.pallas import tpu as pltpu
from jax.experimental.pallas import tpu_sc as plsc

B = 16384
D = 64

_info = plsc.get_sparse_core_info()
_NC, _NS = _info.num_cores, _info.num_subcores
_NW = _NC * _NS
_BPW = B // _NW


def _gather_sc(user_ids, user_table, joke_ids, joke_table):
    mesh = plsc.VectorSubcoreMesh(core_axis_name="c", subcore_axis_name="s")

    @functools.partial(
        pl.kernel,
        mesh=mesh,
        out_type=[
            jax.ShapeDtypeStruct((B, D), jnp.float32),
            jax.ShapeDtypeStruct((B, D), jnp.float32),
        ],
        scratch_types=[
            pltpu.VMEM((_BPW,), jnp.int32),
            pltpu.VMEM((_BPW, D), jnp.float32),
            pltpu.VMEM((_BPW,), jnp.int32),
            pltpu.VMEM((_BPW, D), jnp.float32),
            pltpu.SemaphoreType.DMA,
            pltpu.SemaphoreType.DMA,
        ],
        compiler_params=pltpu.CompilerParams(use_tc_tiling_on_sc=False),
    )
    def gather2(uidx_hbm, utab_hbm, jidx_hbm, jtab_hbm, uout_hbm, jout_hbm,
                uidx_v, urows_v, jidx_v, jrows_v, usem, jsem):
        wid = lax.axis_index("s") * _NC + lax.axis_index("c")
        base = wid * _BPW
        pltpu.sync_copy(uidx_hbm.at[pl.ds(base, _BPW)], uidx_v)
        pltpu.sync_copy(jidx_hbm.at[pl.ds(base, _BPW)], jidx_v)
        cu = pltpu.async_copy(utab_hbm.at[uidx_v], urows_v, usem)
        cj = pltpu.async_copy(jtab_hbm.at[jidx_v], jrows_v, jsem)
        cu.wait()
        cj.wait()
        pltpu.sync_copy(urows_v, uout_hbm.at[pl.ds(base, _BPW)])
        pltpu.sync_copy(jrows_v, jout_hbm.at[pl.ds(base, _BPW)])

    return gather2(user_ids, user_table, joke_ids, joke_table)


def _mlp_body(u_ref, j_ref, w1u_ref, w1j_ref, b1_ref, w2_ref, b2_ref,
              w3_ref, b3_ref, w4_ref, b4_ref, o_ref):
    x = u_ref[...] @ w1u_ref[...] + j_ref[...] @ w1j_ref[...] + b1_ref[...]
    x = jnp.maximum(x, 0.0)
    x = jnp.maximum(x @ w2_ref[...] + b2_ref[...], 0.0)
    x = jnp.maximum(x @ w3_ref[...] + b3_ref[...], 0.0)
    x = jnp.maximum(x @ w4_ref[...] + b4_ref[...], 0.0)
    o_ref[...] = x


def _mlp_tc(user, joke, W1, b1, W2, b2, W3, b3, W4, b4):
    blk = 2048
    grid = (B // blk,)
    w1u = W1[:D]
    w1j = W1[D:]
    full = lambda s: pl.BlockSpec(s, lambda i: (0, 0))
    return pl.pallas_call(
        _mlp_body,
        grid=grid,
        in_specs=[
            pl.BlockSpec((blk, D), lambda i: (i, 0)),
            pl.BlockSpec((blk, D), lambda i: (i, 0)),
            full((D, 32)), full((D, 32)), full((1, 32)),
            full((32, 16)), full((1, 16)),
            full((16, 12)), full((1, 12)),
            full((12, 1)), full((1, 1)),
        ],
        out_specs=pl.BlockSpec((blk, 1), lambda i: (i, 0)),
        out_shape=jax.ShapeDtypeStruct((B, 1), jnp.float32),
        compiler_params=pltpu.CompilerParams(
            dimension_semantics=("arbitrary",),
        ),
    )(user, joke, w1u, w1j, b1.reshape(1, 32), W2, b2.reshape(1, 16),
      W3, b3.reshape(1, 12), W4, b4.reshape(1, 1))


def kernel(user_ids, joke_ids, user_table, joke_table,
           W1, b1, W2, b2, W3, b3, W4, b4):
    uids = user_ids.reshape(B).astype(jnp.int32)
    jids = joke_ids.reshape(B).astype(jnp.int32)
    user, joke = _gather_sc(uids, user_table, jids, joke_table)
    return _mlp_tc(user, joke, W1, b1, W2, b2, W3, b3, W4, b4)


# speedup vs baseline: 1.6498x; 1.6498x over previous
import functools

import jax
import jax.numpy as jnp
from jax import lax
from jax.experimental import pallas as pl
from jax.experimental.pallas import tpu as pltpu
from jax.experimental.pallas import tpu_sc as plsc

B = 16384
D = 64

_info = plsc.get_sparse_core_info()
_NC, _NS = _info.num_cores, _info.num_subcores
_NW = _NC * _NS
_BPW = B // _NW
_CH = 256


def _gather_sc(user_ids, user_table, joke_ids, joke_table):
    mesh = plsc.VectorSubcoreMesh(core_axis_name="c", subcore_axis_name="s")

    @functools.partial(
        pl.kernel,
        mesh=mesh,
        out_type=[
            jax.ShapeDtypeStruct((B, D), jnp.float32),
            jax.ShapeDtypeStruct((B, D), jnp.float32),
        ],
        scratch_types=[
            pltpu.VMEM((_BPW,), jnp.int32),
            pltpu.VMEM((_BPW,), jnp.int32),
            pltpu.VMEM((_CH, D), jnp.float32),
            pltpu.VMEM((_CH, D), jnp.float32),
            pltpu.SemaphoreType.DMA,
            pltpu.SemaphoreType.DMA,
        ],
    )
    def gather2(uidx_hbm, utab_hbm, jidx_hbm, jtab_hbm, uout_hbm, jout_hbm,
                uidx_v, jidx_v, urows_v, jrows_v, usem, jsem):
        wid = lax.axis_index("s") * _NC + lax.axis_index("c")
        base = wid * _BPW
        pltpu.sync_copy(uidx_hbm.at[pl.ds(base, _BPW)], uidx_v)
        pltpu.sync_copy(jidx_hbm.at[pl.ds(base, _BPW)], jidx_v)

        for c in range(_BPW // _CH):
            off = c * _CH

            def body(g, carry):
                row = g * 16
                uvec = uidx_v[pl.ds(off + row, 16)]
                jvec = jidx_v[pl.ds(off + row, 16)]
                for k in range(16):
                    pltpu.async_copy(utab_hbm.at[uvec[k]], urows_v.at[row + k], usem)
                    pltpu.async_copy(jtab_hbm.at[jvec[k]], jrows_v.at[row + k], jsem)
                return carry

            lax.fori_loop(0, _CH // 16, body, 0)
            pltpu.make_async_copy(utab_hbm.at[pl.ds(0, _CH)], urows_v, usem).wait()
            pltpu.make_async_copy(jtab_hbm.at[pl.ds(0, _CH)], jrows_v, jsem).wait()
            pltpu.sync_copy(urows_v, uout_hbm.at[pl.ds(base + off, _CH)])
            pltpu.sync_copy(jrows_v, jout_hbm.at[pl.ds(base + off, _CH)])

    return gather2(user_ids, user_table, joke_ids, joke_table)


def _mlp_body(u_ref, j_ref, w1u_ref, w1j_ref, b1_ref, w2_ref, b2_ref,
              w3_ref, b3_ref, w4_ref, b4_ref, o_ref):
    x = u_ref[...] @ w1u_ref[...] + j_ref[...] @ w1j_ref[...] + b1_ref[...]
    x = jnp.maximum(x, 0.0)
    x = jnp.maximum(x @ w2_ref[...] + b2_ref[...], 0.0)
    x = jnp.maximum(x @ w3_ref[...] + b3_ref[...], 0.0)
    x = jnp.maximum(x @ w4_ref[...] + b4_ref[...], 0.0)
    o_ref[...] = x


def _mlp_tc(user, joke, W1, b1, W2, b2, W3, b3, W4, b4):
    blk = 2048
    grid = (B // blk,)
    w1u = W1[:D]
    w1j = W1[D:]
    full = lambda s: pl.BlockSpec(s, lambda i: (0, 0))
    return pl.pallas_call(
        _mlp_body,
        grid=grid,
        in_specs=[
            pl.BlockSpec((blk, D), lambda i: (i, 0)),
            pl.BlockSpec((blk, D), lambda i: (i, 0)),
            full((D, 32)), full((D, 32)), full((1, 32)),
            full((32, 16)), full((1, 16)),
            full((16, 12)), full((1, 12)),
            full((12, 1)), full((1, 1)),
        ],
        out_specs=pl.BlockSpec((blk, 1), lambda i: (i, 0)),
        out_shape=jax.ShapeDtypeStruct((B, 1), jnp.float32),
        compiler_params=pltpu.CompilerParams(
            dimension_semantics=("arbitrary",),
        ),
    )(user, joke, w1u, w1j, b1.reshape(1, 32), W2, b2.reshape(1, 16),
      W3, b3.reshape(1, 12), W4, b4.reshape(1, 1))


def kernel(user_ids, joke_ids, user_table, joke_table,
           W1, b1, W2, b2, W3, b3, W4, b4):
    uids = user_ids.reshape(B).astype(jnp.int32)
    jids = joke_ids.reshape(B).astype(jnp.int32)
    user, joke = _gather_sc(uids, user_table, jids, joke_table)
    return _mlp_tc(user, joke, W1, b1, W2, b2, W3, b3, W4, b4)


# speedup vs baseline: 2.0199x; 1.2243x over previous
import functools

import jax
import jax.numpy as jnp
from jax import lax
from jax.experimental import pallas as pl
from jax.experimental.pallas import tpu as pltpu
from jax.experimental.pallas import tpu_sc as plsc

B = 16384
D = 64
H1 = 32

_info = plsc.get_sparse_core_info()
_NC, _NS = _info.num_cores, _info.num_subcores
_NW = _NC * _NS
_BPW = B // _NW


def _proj_body(t_ref, w_ref, o_ref):
    o_ref[...] = lax.dot_general(
        t_ref[...], w_ref[...], (((0,), (0,)), ((), ())),
        preferred_element_type=jnp.float32)


def _project(table_t, w):
    rows = table_t.shape[1]
    blk = 8192
    return pl.pallas_call(
        _proj_body,
        grid=(pl.cdiv(rows, blk),),
        in_specs=[
            pl.BlockSpec((D, blk), lambda i: (0, i)),
            pl.BlockSpec((D, H1), lambda i: (0, 0)),
        ],
        out_specs=pl.BlockSpec((blk, H1), lambda i: (i, 0)),
        out_shape=jax.ShapeDtypeStruct((rows, H1), jnp.float32),
        compiler_params=pltpu.CompilerParams(
            dimension_semantics=("arbitrary",),
        ),
    )(table_t, w)


def _gather_sc(ids, ptab):
    mesh = plsc.VectorSubcoreMesh(core_axis_name="c", subcore_axis_name="s")

    @functools.partial(
        pl.kernel,
        mesh=mesh,
        out_type=jax.ShapeDtypeStruct((B, H1), jnp.float32),
        scratch_types=[
            pltpu.VMEM((_BPW,), jnp.int32),
            pltpu.VMEM((_BPW, H1), jnp.float32),
            pltpu.SemaphoreType.DMA,
        ],
    )
    def gather(idx_hbm, tab_hbm, out_hbm, idx_v, rows_v, sem):
        wid = lax.axis_index("s") * _NC + lax.axis_index("c")
        base = wid * _BPW
        pltpu.sync_copy(idx_hbm.at[pl.ds(base, _BPW)], idx_v)

        def body(g, carry):
            row = g * 16
            vec = idx_v[pl.ds(row, 16)]
            for k in range(16):
                pltpu.async_copy(tab_hbm.at[vec[k]], rows_v.at[row + k], sem)
            return carry

        lax.fori_loop(0, _BPW // 16, body, 0)
        pltpu.make_async_copy(tab_hbm.at[pl.ds(0, _BPW)], rows_v, sem).wait()
        pltpu.sync_copy(rows_v, out_hbm.at[pl.ds(base, _BPW)])

    return gather(ids, ptab)


def _tail_body(u_ref, j_ref, b1_ref, w2_ref, b2_ref,
               w3_ref, b3_ref, w4_ref, b4_ref, o_ref):
    x = jnp.maximum(u_ref[...] + j_ref[...] + b1_ref[...], 0.0)
    x = jnp.maximum(x @ w2_ref[...] + b2_ref[...], 0.0)
    x = jnp.maximum(x @ w3_ref[...] + b3_ref[...], 0.0)
    x = jnp.maximum(x @ w4_ref[...] + b4_ref[...], 0.0)
    o_ref[...] = x


def _mlp_tail(gu, gj, b1, W2, b2, W3, b3, W4, b4):
    blk = 2048
    full = lambda s: pl.BlockSpec(s, lambda i: (0, 0))
    return pl.pallas_call(
        _tail_body,
        grid=(B // blk,),
        in_specs=[
            pl.BlockSpec((blk, H1), lambda i: (i, 0)),
            pl.BlockSpec((blk, H1), lambda i: (i, 0)),
            full((1, H1)),
            full((H1, 16)), full((1, 16)),
            full((16, 12)), full((1, 12)),
            full((12, 1)), full((1, 1)),
        ],
        out_specs=pl.BlockSpec((blk, 1), lambda i: (i, 0)),
        out_shape=jax.ShapeDtypeStruct((B, 1), jnp.float32),
        compiler_params=pltpu.CompilerParams(
            dimension_semantics=("arbitrary",),
        ),
    )(gu, gj, b1.reshape(1, H1), W2, b2.reshape(1, 16),
      W3, b3.reshape(1, 12), W4, b4.reshape(1, 1))


def kernel(user_ids, joke_ids, user_table, joke_table,
           W1, b1, W2, b2, W3, b3, W4, b4):
    uids = user_ids.reshape(B).astype(jnp.int32)
    jids = joke_ids.reshape(B).astype(jnp.int32)
    pj = _project(joke_table.T, W1[D:])
    gj = _gather_sc(jids, pj)
    pu = _project(user_table.T, W1[:D])
    gu = _gather_sc(uids, pu)
    return _mlp_tail(gu, gj, b1, W2, b2, W3, b3, W4, b4)


# speedup vs baseline: 2.2454x; 1.1117x over previous
import functools

import jax
import jax.numpy as jnp
from jax import lax
from jax.experimental import pallas as pl
from jax.experimental.pallas import tpu as pltpu
from jax.experimental.pallas import tpu_sc as plsc

B = 16384
D = 64
H1 = 32

_info = plsc.get_sparse_core_info()
_NC, _NS = _info.num_cores, _info.num_subcores
_NW = _NC * _NS
_BPW = B // _NW


def _proj_body(t_ref, w_ref, o_ref):
    o_ref[...] = lax.dot_general(
        t_ref[...], w_ref[...], (((0,), (0,)), ((), ())),
        preferred_element_type=jnp.float32)


def _project(table_t, w):
    rows = table_t.shape[1]
    blk = 32768
    return pl.pallas_call(
        _proj_body,
        grid=(pl.cdiv(rows, blk),),
        in_specs=[
            pl.BlockSpec((D, blk), lambda i: (0, i)),
            pl.BlockSpec((D, H1), lambda i: (0, 0)),
        ],
        out_specs=pl.BlockSpec((blk, H1), lambda i: (i, 0)),
        out_shape=jax.ShapeDtypeStruct((rows, H1), jnp.float32),
        compiler_params=pltpu.CompilerParams(
            dimension_semantics=("arbitrary",),
        ),
    )(table_t, w)


def _gather_sc(ids, ptab):
    mesh = plsc.VectorSubcoreMesh(core_axis_name="c", subcore_axis_name="s")

    @functools.partial(
        pl.kernel,
        mesh=mesh,
        out_type=jax.ShapeDtypeStruct((B, H1), jnp.float32),
        scratch_types=[
            pltpu.VMEM((_BPW,), jnp.int32),
            pltpu.VMEM((_BPW, H1), jnp.float32),
            pltpu.SemaphoreType.DMA,
        ],
    )
    def gather(idx_hbm, tab_hbm, out_hbm, idx_v, rows_v, sem):
        wid = lax.axis_index("s") * _NC + lax.axis_index("c")
        base = wid * _BPW
        pltpu.sync_copy(idx_hbm.at[pl.ds(base, _BPW)], idx_v)

        def body(g, carry):
            row = g * 16
            vec = idx_v[pl.ds(row, 16)]
            for k in range(16):
                pltpu.async_copy(tab_hbm.at[vec[k]], rows_v.at[row + k], sem)
            return carry

        lax.fori_loop(0, _BPW // 16, body, 0)
        pltpu.make_async_copy(tab_hbm.at[pl.ds(0, _BPW)], rows_v, sem).wait()
        pltpu.sync_copy(rows_v, out_hbm.at[pl.ds(base, _BPW)])

    return gather(ids, ptab)


def _tail_body(u_ref, j_ref, b1_ref, w2_ref, b2_ref,
               w3_ref, b3_ref, w4_ref, b4_ref, o_ref):
    x = jnp.maximum(u_ref[...] + j_ref[...] + b1_ref[...], 0.0)
    x = jnp.maximum(x @ w2_ref[...] + b2_ref[...], 0.0)
    x = jnp.maximum(x @ w3_ref[...] + b3_ref[...], 0.0)
    x = jnp.maximum(x @ w4_ref[...] + b4_ref[...], 0.0)
    o_ref[...] = x


def _mlp_tail(gu, gj, b1, W2, b2, W3, b3, W4, b4):
    blk = 2048
    full = lambda s: pl.BlockSpec(s, lambda i: (0, 0))
    return pl.pallas_call(
        _tail_body,
        grid=(B // blk,),
        in_specs=[
            pl.BlockSpec((blk, H1), lambda i: (i, 0)),
            pl.BlockSpec((blk, H1), lambda i: (i, 0)),
            full((1, H1)),
            full((H1, 16)), full((1, 16)),
            full((16, 12)), full((1, 12)),
            full((12, 1)), full((1, 1)),
        ],
        out_specs=pl.BlockSpec((blk, 1), lambda i: (i, 0)),
        out_shape=jax.ShapeDtypeStruct((B, 1), jnp.float32),
        compiler_params=pltpu.CompilerParams(
            dimension_semantics=("arbitrary",),
        ),
    )(gu, gj, b1.reshape(1, H1), W2, b2.reshape(1, 16),
      W3, b3.reshape(1, 12), W4, b4.reshape(1, 1))


def kernel(user_ids, joke_ids, user_table, joke_table,
           W1, b1, W2, b2, W3, b3, W4, b4):
    uids = user_ids.reshape(B).astype(jnp.int32)
    jids = joke_ids.reshape(B).astype(jnp.int32)
    pj = _project(joke_table.T, W1[D:])
    gj = _gather_sc(jids, pj)
    pu = _project(user_table.T, W1[:D])
    gu = _gather_sc(uids, pu)
    return _mlp_tail(gu, gj, b1, W2, b2, W3, b3, W4, b4)
